# baseline (speedup 1.0000x reference)
import functools

import jax
import jax.numpy as jnp
from jax import lax
from jax.experimental import pallas as pl
from jax.experimental.pallas import tpu as pltpu
from jax.experimental.pallas import tpu_sc as plsc

B, L = 16384, 2048
LEDS_NUM = 1024

NC, NS, LANES = 2, 16, 16
NW = NC * NS

ROWS_W = B // NW
R = 8
STEPS = ROWS_W // R


def _make_sc_call():
    mesh = plsc.VectorSubcoreMesh(core_axis_name="c", subcore_axis_name="s")

    @functools.partial(
        pl.kernel,
        mesh=mesh,
        out_type=jax.ShapeDtypeStruct((B, L), jnp.float32),
        scratch_types=[
            pltpu.VMEM((LEDS_NUM,), jnp.float32),
            [pltpu.VMEM((R, L), jnp.float32)] * 2,
            [pltpu.VMEM((R, L), jnp.int32)] * 2,
            [pltpu.VMEM((R, L), jnp.float32)] * 2,
            [pltpu.SemaphoreType.DMA] * 2,
            [pltpu.SemaphoreType.DMA] * 2,
            [pltpu.SemaphoreType.DMA] * 2,
        ],
        compiler_params=pltpu.CompilerParams(needs_layout_passes=False),
    )
    def sc_kernel(x_hbm, idx_hbm, coeff_hbm, out_hbm,
                  table_v, x_v, idx_v, out_v, sx, si, so):
        wid = lax.axis_index("s") * NC + lax.axis_index("c")

        pltpu.sync_copy(coeff_hbm, table_v)

        def start_in(step, b):
            row = (step * NW + wid) * R
            pltpu.async_copy(x_hbm.at[pl.ds(row, R)], x_v[b], sx[b])
            pltpu.async_copy(idx_hbm.at[pl.ds(row, R)], idx_v[b], si[b])

        def wait_in(step, b):
            row = (step * NW + wid) * R
            pltpu.make_async_copy(x_hbm.at[pl.ds(row, R)], x_v[b], sx[b]).wait()
            pltpu.make_async_copy(idx_hbm.at[pl.ds(row, R)], idx_v[b], si[b]).wait()

        def start_out(step, b):
            row = (step * NW + wid) * R
            pltpu.async_copy(out_v[b], out_hbm.at[pl.ds(row, R)], so[b])

        def wait_out(step, b):
            row = (step * NW + wid) * R
            pltpu.make_async_copy(out_v[b], out_hbm.at[pl.ds(row, R)], so[b]).wait()

        def compute(b):
            @plsc.parallel_loop(0, L, step=LANES, unroll=4)
            def _(e):
                sl = pl.ds(e, LANES)
                for r in range(R):
                    c = plsc.load_gather(table_v, [idx_v[b][r, sl]])
                    out_v[b][r, sl] = x_v[b][r, sl] * c

        start_in(0, 0)
        start_in(1, 1)

        def pair(p, _):
            for b in range(2):
                s = 2 * p + b
                wait_in(s, b)

                @pl.when(p >= 1)
                def _():
                    wait_out(s - 2, b)

                compute(b)
                start_out(s, b)

                @pl.when(s + 2 < STEPS)
                def _():
                    start_in(s + 2, b)
            return 0

        lax.fori_loop(0, STEPS // 2, pair, 0)
        wait_out(STEPS - 2, 0)
        wait_out(STEPS - 1, 1)

    return sc_kernel


_sc_call = _make_sc_call()


@jax.jit
def kernel(x, idx, coeff):
    return _sc_call(x, idx.astype(jnp.int32), coeff)


# speedup vs baseline: 1.0070x; 1.0070x over previous
import functools

import jax
import jax.numpy as jnp
from jax import lax
from jax.experimental import pallas as pl
from jax.experimental.pallas import tpu as pltpu
from jax.experimental.pallas import tpu_sc as plsc

B, L = 16384, 2048
LEDS_NUM = 1024

NC, NS, LANES = 2, 16, 16
NW = NC * NS

ROWS_W = B // NW
R = 8
STEPS = ROWS_W // R


def _make_sc_call():
    mesh = plsc.VectorSubcoreMesh(core_axis_name="c", subcore_axis_name="s")

    @functools.partial(
        pl.kernel,
        mesh=mesh,
        out_type=jax.ShapeDtypeStruct((B, L), jnp.float32),
        scratch_types=[
            pltpu.VMEM((LEDS_NUM,), jnp.float32),
            [pltpu.VMEM((R, L), jnp.float32)] * 2,
            [pltpu.VMEM((R, L), jnp.int32)] * 2,
            [pltpu.VMEM((R, L), jnp.float32)] * 2,
            [pltpu.SemaphoreType.DMA] * 2,
            [pltpu.SemaphoreType.DMA] * 2,
            [pltpu.SemaphoreType.DMA] * 2,
        ],
        compiler_params=pltpu.CompilerParams(needs_layout_passes=False),
    )
    def sc_kernel(x_hbm, idx_hbm, coeff_hbm, out_hbm,
                  table_v, x_v, idx_v, out_v, sx, si, so):
        wid = lax.axis_index("s") * NC + lax.axis_index("c")

        def start_in(step, b):
            row = (step * NW + wid) * R
            pltpu.async_copy(x_hbm.at[pl.ds(row, R)], x_v[b], sx[b])
            pltpu.async_copy(idx_hbm.at[pl.ds(row, R)], idx_v[b], si[b])

        def wait_in(step, b):
            row = (step * NW + wid) * R
            pltpu.make_async_copy(x_hbm.at[pl.ds(row, R)], x_v[b], sx[b]).wait()
            pltpu.make_async_copy(idx_hbm.at[pl.ds(row, R)], idx_v[b], si[b]).wait()

        def start_out(step, b):
            row = (step * NW + wid) * R
            pltpu.async_copy(out_v[b], out_hbm.at[pl.ds(row, R)], so[b])

        def wait_out(step, b):
            row = (step * NW + wid) * R
            pltpu.make_async_copy(out_v[b], out_hbm.at[pl.ds(row, R)], so[b]).wait()

        def compute(b):
            @plsc.parallel_loop(0, L, step=LANES, unroll=4)
            def _(e):
                sl = pl.ds(e, LANES)
                for r in range(R):
                    c = plsc.load_gather(table_v, [idx_v[b][r, sl]])
                    out_v[b][r, sl] = x_v[b][r, sl] * c

        start_in(0, 0)
        start_in(1, 1)
        pltpu.sync_copy(coeff_hbm, table_v)

        def pair(p, _):
            for b in range(2):
                s = 2 * p + b
                wait_in(s, b)

                @pl.when(p >= 1)
                def _():
                    wait_out(s - 2, b)

                compute(b)
                start_out(s, b)

                @pl.when(s + 2 < STEPS)
                def _():
                    start_in(s + 2, b)
            return 0

        lax.fori_loop(0, STEPS // 2, pair, 0)
        wait_out(STEPS - 2, 0)
        wait_out(STEPS - 1, 1)

    return sc_kernel


_sc_call = _make_sc_call()


@jax.jit
def kernel(x, idx, coeff):
    return _sc_call(x, idx.astype(jnp.int32), coeff)


# speedup vs baseline: 1.0106x; 1.0036x over previous
import functools

import jax
import jax.numpy as jnp
from jax import lax
from jax.experimental import pallas as pl
from jax.experimental.pallas import tpu as pltpu
from jax.experimental.pallas import tpu_sc as plsc

B, L = 16384, 2048
LEDS_NUM = 1024

NC, NS, LANES = 2, 16, 16
NW = NC * NS

ROWS_W = B // NW
R = 8
STEPS = ROWS_W // R


def _make_sc_call():
    mesh = plsc.VectorSubcoreMesh(core_axis_name="c", subcore_axis_name="s")

    @functools.partial(
        pl.kernel,
        mesh=mesh,
        out_type=jax.ShapeDtypeStruct((B, L), jnp.float32),
        scratch_types=[
            pltpu.VMEM((LEDS_NUM,), jnp.float32),
            [pltpu.VMEM((R, L), jnp.float32)] * 2,
            [pltpu.VMEM((R, L), jnp.int32)] * 2,
            [pltpu.VMEM((R, L), jnp.float32)] * 2,
            [pltpu.SemaphoreType.DMA] * 2,
            [pltpu.SemaphoreType.DMA] * 2,
            [pltpu.SemaphoreType.DMA] * 2,
        ],
        compiler_params=pltpu.CompilerParams(needs_layout_passes=False),
    )
    def sc_kernel(x_hbm, idx_hbm, coeff_hbm, out_hbm,
                  table_v, x_v, idx_v, out_v, sx, si, so):
        wid = lax.axis_index("c") * NS + lax.axis_index("s")

        def start_in(step, b):
            row = (step * NW + wid) * R
            pltpu.async_copy(x_hbm.at[pl.ds(row, R)], x_v[b], sx[b])
            pltpu.async_copy(idx_hbm.at[pl.ds(row, R)], idx_v[b], si[b])

        def wait_in(step, b):
            row = (step * NW + wid) * R
            pltpu.make_async_copy(x_hbm.at[pl.ds(row, R)], x_v[b], sx[b]).wait()
            pltpu.make_async_copy(idx_hbm.at[pl.ds(row, R)], idx_v[b], si[b]).wait()

        def start_out(step, b):
            row = (step * NW + wid) * R
            pltpu.async_copy(out_v[b], out_hbm.at[pl.ds(row, R)], so[b])

        def wait_out(step, b):
            row = (step * NW + wid) * R
            pltpu.make_async_copy(out_v[b], out_hbm.at[pl.ds(row, R)], so[b]).wait()

        def compute(b):
            @plsc.parallel_loop(0, L, step=LANES, unroll=4)
            def _(e):
                sl = pl.ds(e, LANES)
                for r in range(R):
                    c = plsc.load_gather(table_v, [idx_v[b][r, sl]])
                    out_v[b][r, sl] = x_v[b][r, sl] * c

        start_in(0, 0)
        start_in(1, 1)
        pltpu.sync_copy(coeff_hbm, table_v)

        def pair(p, _):
            for b in range(2):
                s = 2 * p + b
                wait_in(s, b)

                @pl.when(p >= 1)
                def _():
                    wait_out(s - 2, b)

                compute(b)
                start_out(s, b)

                @pl.when(s + 2 < STEPS)
                def _():
                    start_in(s + 2, b)
            return 0

        lax.fori_loop(0, STEPS // 2, pair, 0)
        wait_out(STEPS - 2, 0)
        wait_out(STEPS - 1, 1)

    return sc_kernel


_sc_call = _make_sc_call()


@jax.jit
def kernel(x, idx, coeff):
    return _sc_call(x, idx.astype(jnp.int32), coeff)
